# baseline (speedup 1.0000x reference)
import jax
import jax.numpy as jnp
from jax.experimental import pallas as pl
from jax.experimental.pallas import tpu as pltpu

DIM = 1024
INTER = 1024
NR = 15
TOPK = 7
LANES = 128
SQRT1_2 = 0.7071067811865476


def _gelu(h):
    return 0.5 * h * (1.0 + jax.lax.erf(h * SQRT1_2))


def _router_kernel(x_ref, wr_ref, br_ref, w1s_ref, b1s_ref, w2s_ref, b2s_ref,
                   g_ref, xbf_ref, oinit_ref):
    x = x_ref[...]
    logits = jnp.dot(x, wr_ref[...], preferred_element_type=jnp.float32)
    logits = logits + br_ref[...]
    lane = jax.lax.broadcasted_iota(jnp.int32, logits.shape, 1)
    valid = lane < NR
    logits = jnp.where(valid, logits, -1e30)
    m = jnp.max(logits, axis=1, keepdims=True)
    ex = jnp.where(valid, jnp.exp(logits - m), 0.0)
    aff = ex / jnp.sum(ex, axis=1, keepdims=True)
    work = aff
    gates = jnp.zeros_like(aff)
    for _ in range(TOPK):
        idx = jnp.argmax(work, axis=1)
        sel = lane == idx[:, None]
        gates = jnp.where(sel, aff, gates)
        work = jnp.where(sel, -1.0, work)
    g_ref[...] = gates

    xb = x.astype(jnp.bfloat16)
    xbf_ref[...] = xb
    h = jnp.dot(xb, w1s_ref[...].astype(jnp.bfloat16),
                preferred_element_type=jnp.float32) + b1s_ref[...]
    h = _gelu(h)
    eo = jnp.dot(h.astype(jnp.bfloat16), w2s_ref[...].astype(jnp.bfloat16),
                 preferred_element_type=jnp.float32) + b2s_ref[...]
    oinit_ref[...] = x + eo


def _expert_kernel(oi_ref, g_ref, xbf_ref, w1_ref, b1_ref, w2_ref, b2_ref,
                   out_ref):
    e = pl.program_id(0)
    xb = xbf_ref[...]
    JT = 2
    JW = INTER // JT
    eo = b2_ref[0]
    for j in range(JT):
        hj = jnp.dot(xb, w1_ref[0, :, j * JW:(j + 1) * JW].astype(jnp.bfloat16),
                     preferred_element_type=jnp.float32)
        hj = hj + b1_ref[0, :, j * JW:(j + 1) * JW]
        hj = _gelu(hj)
        eo = eo + jnp.dot(hj.astype(jnp.bfloat16),
                          w2_ref[0, j * JW:(j + 1) * JW, :].astype(jnp.bfloat16),
                          preferred_element_type=jnp.float32)
    lane = jax.lax.broadcasted_iota(jnp.int32, g_ref.shape, 1)
    g = jnp.sum(jnp.where(lane == e, g_ref[...], 0.0), axis=1, keepdims=True)
    contrib = eo * g

    @pl.when(e == 0)
    def _first():
        out_ref[...] = oi_ref[...] + contrib

    @pl.when(e != 0)
    def _rest():
        out_ref[...] += contrib


def kernel(x, W1s, b1s, W2s, b2s, W1r, b1r, W2r, b2r, Wr, br):
    B, S, D = x.shape
    x2 = x.reshape(S, D)

    wr_pad = jnp.zeros((D, LANES), jnp.float32).at[:, :NR].set(Wr)
    br_pad = jnp.zeros((1, LANES), jnp.float32).at[0, :NR].set(br)

    gates, xbf, out_init = pl.pallas_call(
        _router_kernel,
        out_shape=(
            jax.ShapeDtypeStruct((S, LANES), jnp.float32),
            jax.ShapeDtypeStruct((S, D), jnp.bfloat16),
            jax.ShapeDtypeStruct((S, D), jnp.float32),
        ),
    )(x2, wr_pad, br_pad, W1s, b1s.reshape(1, INTER), W2s, b2s.reshape(1, D))

    out = pl.pallas_call(
        _expert_kernel,
        grid=(NR,),
        in_specs=[
            pl.BlockSpec((S, D), lambda e: (0, 0)),
            pl.BlockSpec((S, LANES), lambda e: (0, 0)),
            pl.BlockSpec((S, D), lambda e: (0, 0)),
            pl.BlockSpec((1, D, INTER), lambda e: (e, 0, 0)),
            pl.BlockSpec((1, 1, INTER), lambda e: (e, 0, 0)),
            pl.BlockSpec((1, INTER, D), lambda e: (e, 0, 0)),
            pl.BlockSpec((1, 1, D), lambda e: (e, 0, 0)),
        ],
        out_specs=pl.BlockSpec((S, D), lambda e: (0, 0)),
        out_shape=jax.ShapeDtypeStruct((S, D), jnp.float32),
        compiler_params=pltpu.CompilerParams(
            dimension_semantics=("arbitrary",),
        ),
    )(out_init, gates, xbf, W1r, b1r.reshape(NR, 1, INTER),
      W2r, b2r.reshape(NR, 1, D))

    return out.reshape(B, S, D)


# speedup vs baseline: 1.0232x; 1.0232x over previous
import jax
import jax.numpy as jnp
from jax.experimental import pallas as pl
from jax.experimental.pallas import tpu as pltpu

DIM = 1024
INTER = 1024
NR = 15
TOPK = 7
LANES = 128
SQRT1_2 = 0.7071067811865476


def _gelu(h):
    return 0.5 * h * (1.0 + jax.lax.erf(h * SQRT1_2))


def _router_kernel(x_ref, wr_ref, br_ref, w1s_ref, b1s_ref, w2s_ref, b2s_ref,
                   g_ref, xbf_ref, oinit_ref):
    x = x_ref[...]
    logits = jnp.dot(x, wr_ref[...], preferred_element_type=jnp.float32)
    logits = logits + br_ref[...]
    lane = jax.lax.broadcasted_iota(jnp.int32, logits.shape, 1)
    valid = lane < NR
    logits = jnp.where(valid, logits, -1e30)
    m = jnp.max(logits, axis=1, keepdims=True)
    ex = jnp.where(valid, jnp.exp(logits - m), 0.0)
    aff = ex / jnp.sum(ex, axis=1, keepdims=True)
    work = aff
    gates = jnp.zeros_like(aff)
    for _ in range(TOPK):
        idx = jnp.argmax(work, axis=1)
        sel = lane == idx[:, None]
        gates = jnp.where(sel, aff, gates)
        work = jnp.where(sel, -1.0, work)
    g_ref[...] = gates

    xb = x.astype(jnp.bfloat16)
    xbf_ref[...] = xb
    h = jnp.dot(xb, w1s_ref[...].astype(jnp.bfloat16),
                preferred_element_type=jnp.float32) + b1s_ref[...]
    h = _gelu(h)
    eo = jnp.dot(h.astype(jnp.bfloat16), w2s_ref[...].astype(jnp.bfloat16),
                 preferred_element_type=jnp.float32) + b2s_ref[...]
    oinit_ref[...] = x + eo


def _expert_kernel(oi_ref, g_ref, xbf_ref, w1_ref, b1_ref, w2_ref, b2_ref,
                   out_ref):
    e = pl.program_id(0)
    h = jnp.dot(xbf_ref[...], w1_ref[0].astype(jnp.bfloat16),
                preferred_element_type=jnp.float32) + b1_ref[0]
    h = _gelu(h)
    eo = jnp.dot(h.astype(jnp.bfloat16), w2_ref[0].astype(jnp.bfloat16),
                 preferred_element_type=jnp.float32) + b2_ref[0]
    lane = jax.lax.broadcasted_iota(jnp.int32, g_ref.shape, 1)
    g = jnp.sum(jnp.where(lane == e, g_ref[...], 0.0), axis=1, keepdims=True)
    contrib = eo * g

    @pl.when(e == 0)
    def _first():
        out_ref[...] = oi_ref[...] + contrib

    @pl.when(e != 0)
    def _rest():
        out_ref[...] += contrib


def kernel(x, W1s, b1s, W2s, b2s, W1r, b1r, W2r, b2r, Wr, br):
    B, S, D = x.shape
    x2 = x.reshape(S, D)

    wr_pad = jnp.zeros((D, LANES), jnp.float32).at[:, :NR].set(Wr)
    br_pad = jnp.zeros((1, LANES), jnp.float32).at[0, :NR].set(br)

    BR = 256
    gates, xbf, out_init = pl.pallas_call(
        _router_kernel,
        grid=(S // BR,),
        in_specs=[
            pl.BlockSpec((BR, D), lambda i: (i, 0)),
            pl.BlockSpec((D, LANES), lambda i: (0, 0)),
            pl.BlockSpec((1, LANES), lambda i: (0, 0)),
            pl.BlockSpec((D, INTER), lambda i: (0, 0)),
            pl.BlockSpec((1, INTER), lambda i: (0, 0)),
            pl.BlockSpec((INTER, D), lambda i: (0, 0)),
            pl.BlockSpec((1, D), lambda i: (0, 0)),
        ],
        out_specs=(
            pl.BlockSpec((BR, LANES), lambda i: (i, 0)),
            pl.BlockSpec((BR, D), lambda i: (i, 0)),
            pl.BlockSpec((BR, D), lambda i: (i, 0)),
        ),
        out_shape=(
            jax.ShapeDtypeStruct((S, LANES), jnp.float32),
            jax.ShapeDtypeStruct((S, D), jnp.bfloat16),
            jax.ShapeDtypeStruct((S, D), jnp.float32),
        ),
        compiler_params=pltpu.CompilerParams(
            dimension_semantics=("arbitrary",),
        ),
    )(x2, wr_pad, br_pad, W1s, b1s.reshape(1, INTER), W2s, b2s.reshape(1, D))

    out = pl.pallas_call(
        _expert_kernel,
        grid=(NR,),
        in_specs=[
            pl.BlockSpec((S, D), lambda e: (0, 0)),
            pl.BlockSpec((S, LANES), lambda e: (0, 0)),
            pl.BlockSpec((S, D), lambda e: (0, 0)),
            pl.BlockSpec((1, D, INTER), lambda e: (e, 0, 0)),
            pl.BlockSpec((1, 1, INTER), lambda e: (e, 0, 0)),
            pl.BlockSpec((1, INTER, D), lambda e: (e, 0, 0)),
            pl.BlockSpec((1, 1, D), lambda e: (e, 0, 0)),
        ],
        out_specs=pl.BlockSpec((S, D), lambda e: (0, 0)),
        out_shape=jax.ShapeDtypeStruct((S, D), jnp.float32),
        compiler_params=pltpu.CompilerParams(
            dimension_semantics=("arbitrary",),
        ),
    )(out_init, gates, xbf, W1r, b1r.reshape(NR, 1, INTER),
      W2r, b2r.reshape(NR, 1, D))

    return out.reshape(B, S, D)


# speedup vs baseline: 1.0646x; 1.0405x over previous
import jax
import jax.numpy as jnp
from jax.experimental import pallas as pl
from jax.experimental.pallas import tpu as pltpu

DIM = 1024
INTER = 1024
NR = 15
TOPK = 7
LANES = 128
SQRT1_2 = 0.7071067811865476


def _gelu(h):
    return 0.5 * h * (1.0 + jax.lax.erf(h * SQRT1_2))


def _router_kernel(x_ref, wr_ref, br_ref, w1s_ref, b1s_ref, w2s_ref, b2s_ref,
                   g_ref, xbf_ref, oinit_ref):
    x = x_ref[...]
    logits = jnp.dot(x, wr_ref[...], preferred_element_type=jnp.float32)
    logits = logits + br_ref[...]
    lane = jax.lax.broadcasted_iota(jnp.int32, logits.shape, 1)
    valid = lane < NR
    logits = jnp.where(valid, logits, -1e30)
    m = jnp.max(logits, axis=1, keepdims=True)
    ex = jnp.where(valid, jnp.exp(logits - m), 0.0)
    aff = ex / jnp.sum(ex, axis=1, keepdims=True)
    work = aff
    gates = jnp.zeros_like(aff)
    for _ in range(TOPK):
        idx = jnp.argmax(work, axis=1)
        sel = lane == idx[:, None]
        gates = jnp.where(sel, aff, gates)
        work = jnp.where(sel, -1.0, work)
    g_ref[...] = gates

    xb = x.astype(jnp.bfloat16)
    xbf_ref[...] = xb
    h = jnp.dot(xb, w1s_ref[...].astype(jnp.bfloat16),
                preferred_element_type=jnp.float32) + b1s_ref[...]
    h = _gelu(h)
    eo = jnp.dot(h.astype(jnp.bfloat16), w2s_ref[...].astype(jnp.bfloat16),
                 preferred_element_type=jnp.float32) + b2s_ref[...]
    oinit_ref[...] = x + eo


def _expert_kernel(oi_ref, g_ref, xbf_ref, w1_ref, b1_ref, w2_ref, b2_ref,
                   out_ref):
    e = pl.program_id(0)
    h = jnp.dot(xbf_ref[...], w1_ref[0].astype(jnp.bfloat16),
                preferred_element_type=jnp.float32) + b1_ref[0]
    hb = h.astype(jnp.bfloat16)
    hb = (0.5 * hb * (1.0 + jax.lax.erf(hb * jnp.bfloat16(SQRT1_2))))
    eo = jnp.dot(hb, w2_ref[0].astype(jnp.bfloat16),
                 preferred_element_type=jnp.float32) + b2_ref[0]
    lane = jax.lax.broadcasted_iota(jnp.int32, g_ref.shape, 1)
    g = jnp.sum(jnp.where(lane == e, g_ref[...], 0.0), axis=1, keepdims=True)
    contrib = eo * g

    @pl.when(e == 0)
    def _first():
        out_ref[...] = oi_ref[...] + contrib

    @pl.when(e != 0)
    def _rest():
        out_ref[...] += contrib


def kernel(x, W1s, b1s, W2s, b2s, W1r, b1r, W2r, b2r, Wr, br):
    B, S, D = x.shape
    x2 = x.reshape(S, D)

    wr_pad = jnp.zeros((D, LANES), jnp.float32).at[:, :NR].set(Wr)
    br_pad = jnp.zeros((1, LANES), jnp.float32).at[0, :NR].set(br)

    BR = 256
    gates, xbf, out_init = pl.pallas_call(
        _router_kernel,
        grid=(S // BR,),
        in_specs=[
            pl.BlockSpec((BR, D), lambda i: (i, 0)),
            pl.BlockSpec((D, LANES), lambda i: (0, 0)),
            pl.BlockSpec((1, LANES), lambda i: (0, 0)),
            pl.BlockSpec((D, INTER), lambda i: (0, 0)),
            pl.BlockSpec((1, INTER), lambda i: (0, 0)),
            pl.BlockSpec((INTER, D), lambda i: (0, 0)),
            pl.BlockSpec((1, D), lambda i: (0, 0)),
        ],
        out_specs=(
            pl.BlockSpec((BR, LANES), lambda i: (i, 0)),
            pl.BlockSpec((BR, D), lambda i: (i, 0)),
            pl.BlockSpec((BR, D), lambda i: (i, 0)),
        ),
        out_shape=(
            jax.ShapeDtypeStruct((S, LANES), jnp.float32),
            jax.ShapeDtypeStruct((S, D), jnp.bfloat16),
            jax.ShapeDtypeStruct((S, D), jnp.float32),
        ),
        compiler_params=pltpu.CompilerParams(
            dimension_semantics=("arbitrary",),
        ),
    )(x2, wr_pad, br_pad, W1s, b1s.reshape(1, INTER), W2s, b2s.reshape(1, D))

    out = pl.pallas_call(
        _expert_kernel,
        grid=(NR,),
        in_specs=[
            pl.BlockSpec((S, D), lambda e: (0, 0)),
            pl.BlockSpec((S, LANES), lambda e: (0, 0)),
            pl.BlockSpec((S, D), lambda e: (0, 0)),
            pl.BlockSpec((1, D, INTER), lambda e: (e, 0, 0)),
            pl.BlockSpec((1, 1, INTER), lambda e: (e, 0, 0)),
            pl.BlockSpec((1, INTER, D), lambda e: (e, 0, 0)),
            pl.BlockSpec((1, 1, D), lambda e: (e, 0, 0)),
        ],
        out_specs=pl.BlockSpec((S, D), lambda e: (0, 0)),
        out_shape=jax.ShapeDtypeStruct((S, D), jnp.float32),
        compiler_params=pltpu.CompilerParams(
            dimension_semantics=("arbitrary",),
        ),
    )(out_init, gates, xbf, W1r, b1r.reshape(NR, 1, INTER),
      W2r, b2r.reshape(NR, 1, D))

    return out.reshape(B, S, D)


# speedup vs baseline: 1.0760x; 1.0107x over previous
import jax
import jax.numpy as jnp
from jax.experimental import pallas as pl
from jax.experimental.pallas import tpu as pltpu

DIM = 1024
INTER = 1024
NR = 15
TOPK = 7
LANES = 128
SQRT1_2 = 0.7071067811865476


def _gelu(h):
    return 0.5 * h * (1.0 + jax.lax.erf(h * SQRT1_2))


def _router_kernel(x_ref, wr_ref, br_ref, w1s_ref, b1s_ref, w2s_ref, b2s_ref,
                   g_ref, xbf_ref, oinit_ref):
    x = x_ref[...]
    logits = jnp.dot(x, wr_ref[...], preferred_element_type=jnp.float32)
    logits = logits + br_ref[...]
    lane = jax.lax.broadcasted_iota(jnp.int32, logits.shape, 1)
    valid = lane < NR
    logits = jnp.where(valid, logits, -1e30)
    m = jnp.max(logits, axis=1, keepdims=True)
    ex = jnp.where(valid, jnp.exp(logits - m), 0.0)
    aff = ex / jnp.sum(ex, axis=1, keepdims=True)
    work = aff
    gates = jnp.zeros_like(aff)
    for _ in range(TOPK):
        idx = jnp.argmax(work, axis=1)
        sel = lane == idx[:, None]
        gates = jnp.where(sel, aff, gates)
        work = jnp.where(sel, -1.0, work)
    g_ref[...] = gates

    xb = x.astype(jnp.bfloat16)
    xbf_ref[...] = xb
    h = jnp.dot(xb, w1s_ref[...].astype(jnp.bfloat16),
                preferred_element_type=jnp.float32) + b1s_ref[...]
    hb = h.astype(jnp.bfloat16)
    hb = 0.5 * hb * (1.0 + jax.lax.erf(hb * jnp.bfloat16(SQRT1_2)))
    eo = jnp.dot(hb, w2s_ref[...].astype(jnp.bfloat16),
                 preferred_element_type=jnp.float32) + b2s_ref[...]
    oinit_ref[...] = x + eo


def _expert_kernel(oi_ref, g_ref, xbf_ref, w1_ref, b1_ref, w2_ref, b2_ref,
                   out_ref):
    e = pl.program_id(0)
    h = jnp.dot(xbf_ref[...], w1_ref[0].astype(jnp.bfloat16),
                preferred_element_type=jnp.float32) + b1_ref[0]
    hb = h.astype(jnp.bfloat16)
    hb = (0.5 * hb * (1.0 + jax.lax.erf(hb * jnp.bfloat16(SQRT1_2))))
    eo = jnp.dot(hb, w2_ref[0].astype(jnp.bfloat16),
                 preferred_element_type=jnp.float32) + b2_ref[0]
    lane = jax.lax.broadcasted_iota(jnp.int32, g_ref.shape, 1)
    g = jnp.sum(jnp.where(lane == e, g_ref[...], 0.0), axis=1, keepdims=True)
    contrib = eo * g

    @pl.when(e == 0)
    def _first():
        out_ref[...] = oi_ref[...] + contrib

    @pl.when(e != 0)
    def _rest():
        out_ref[...] += contrib


def kernel(x, W1s, b1s, W2s, b2s, W1r, b1r, W2r, b2r, Wr, br):
    B, S, D = x.shape
    x2 = x.reshape(S, D)

    wr_pad = jnp.zeros((D, LANES), jnp.float32).at[:, :NR].set(Wr)
    br_pad = jnp.zeros((1, LANES), jnp.float32).at[0, :NR].set(br)

    BR = 256
    gates, xbf, out_init = pl.pallas_call(
        _router_kernel,
        grid=(S // BR,),
        in_specs=[
            pl.BlockSpec((BR, D), lambda i: (i, 0)),
            pl.BlockSpec((D, LANES), lambda i: (0, 0)),
            pl.BlockSpec((1, LANES), lambda i: (0, 0)),
            pl.BlockSpec((D, INTER), lambda i: (0, 0)),
            pl.BlockSpec((1, INTER), lambda i: (0, 0)),
            pl.BlockSpec((INTER, D), lambda i: (0, 0)),
            pl.BlockSpec((1, D), lambda i: (0, 0)),
        ],
        out_specs=(
            pl.BlockSpec((BR, LANES), lambda i: (i, 0)),
            pl.BlockSpec((BR, D), lambda i: (i, 0)),
            pl.BlockSpec((BR, D), lambda i: (i, 0)),
        ),
        out_shape=(
            jax.ShapeDtypeStruct((S, LANES), jnp.float32),
            jax.ShapeDtypeStruct((S, D), jnp.bfloat16),
            jax.ShapeDtypeStruct((S, D), jnp.float32),
        ),
        compiler_params=pltpu.CompilerParams(
            dimension_semantics=("arbitrary",),
        ),
    )(x2, wr_pad, br_pad, W1s, b1s.reshape(1, INTER), W2s, b2s.reshape(1, D))

    out = pl.pallas_call(
        _expert_kernel,
        grid=(NR,),
        in_specs=[
            pl.BlockSpec((S, D), lambda e: (0, 0)),
            pl.BlockSpec((S, LANES), lambda e: (0, 0)),
            pl.BlockSpec((S, D), lambda e: (0, 0)),
            pl.BlockSpec((1, D, INTER), lambda e: (e, 0, 0)),
            pl.BlockSpec((1, 1, INTER), lambda e: (e, 0, 0)),
            pl.BlockSpec((1, INTER, D), lambda e: (e, 0, 0)),
            pl.BlockSpec((1, 1, D), lambda e: (e, 0, 0)),
        ],
        out_specs=pl.BlockSpec((S, D), lambda e: (0, 0)),
        out_shape=jax.ShapeDtypeStruct((S, D), jnp.float32),
        compiler_params=pltpu.CompilerParams(
            dimension_semantics=("arbitrary",),
        ),
    )(out_init, gates, xbf, W1r, b1r.reshape(NR, 1, INTER),
      W2r, b2r.reshape(NR, 1, D))

    return out.reshape(B, S, D)
